# baseline (speedup 1.0000x reference)
import functools

import jax
import jax.numpy as jnp
from jax.experimental import pallas as pl
from jax.experimental.pallas import tpu as pltpu

_L = 2
_E = 8
_H = 768
_TB = 256


def _moe_layer_body(x_ref, rw_ref, rb_ref, ew_ref, eb_ref, o_ref):
    x = x_ref[...]
    logits = jax.lax.dot_general(x, rw_ref[...], (((1,), (1,)), ((), ())))
    logits = logits + rb_ref[...]
    iota = jax.lax.broadcasted_iota(jnp.int32, (_TB, _E), 1)
    m1 = jnp.max(logits, axis=1, keepdims=True)
    e1 = jnp.min(jnp.where(logits == m1, iota, _E), axis=1, keepdims=True)
    neg = jnp.float32(-3.0e38)
    l2 = jnp.where(iota == e1, neg, logits)
    m2 = jnp.max(l2, axis=1, keepdims=True)
    e2 = jnp.min(jnp.where(l2 == m2, iota, _E), axis=1, keepdims=True)
    acc = jnp.zeros((_TB, _H), jnp.float32)
    for e in range(_E):
        sel = (e1 == e) | (e2 == e)
        y = jax.lax.dot_general(x, ew_ref[e], (((1,), (1,)), ((), ())))
        y = y + eb_ref[e][None, :]
        acc = acc + jnp.where(sel, y, 0.0)
    o_ref[...] = acc


def _moe_layer(x, rw, rb, ew, eb):
    n = x.shape[0]
    return pl.pallas_call(
        _moe_layer_body,
        grid=(n // _TB,),
        in_specs=[
            pl.BlockSpec((_TB, _H), lambda i: (i, 0)),
            pl.BlockSpec((_E, _H), lambda i: (0, 0)),
            pl.BlockSpec((1, _E), lambda i: (0, 0)),
            pl.BlockSpec((_E, _H, _H), lambda i: (0, 0, 0)),
            pl.BlockSpec((_E, _H), lambda i: (0, 0)),
        ],
        out_specs=pl.BlockSpec((_TB, _H), lambda i: (i, 0)),
        out_shape=jax.ShapeDtypeStruct((n, _H), jnp.float32),
    )(x, rw, rb.reshape(1, _E), ew, eb)


def kernel(input_ids, router_w, router_b, expert_w, expert_b):
    bsz, seq = input_ids.shape
    hs = jax.random.normal(jax.random.key(42), (bsz, seq, _H), dtype=jnp.float32)
    x = hs.reshape(bsz * seq, _H)
    for l in range(_L):
        x = _moe_layer(x, router_w[l], router_b[l], expert_w[l], expert_b[l])
    return x.reshape(bsz, seq, _H)
